# Initial kernel scaffold; baseline (speedup 1.0000x reference)
import jax
import jax.numpy as jnp
from jax.experimental import pallas as pl


def kernel(action, label):
    raise NotImplementedError("write your pallas kernel here")


# speedup vs baseline: 9.2086x; 9.2086x over previous
import functools

import jax
import jax.numpy as jnp
from jax.experimental import pallas as pl
from jax.experimental.pallas import tpu as pltpu

_B = 16384
_A = 128
_BLOCK_ROWS = 2048


def _loss_kernel(label_ref, x_ref, out_ref, acc_ref):
    i = pl.program_id(0)

    @pl.when(i == 0)
    def _init():
        acc_ref[0] = 0.0
        acc_ref[1] = 0.0

    factor = jnp.where(label_ref[0] == 1, jnp.float32(1.25), jnp.float32(0.9))
    x = x_ref[...]
    rowsum = jnp.sum(x, axis=1)
    rowmax = jnp.max(x, axis=1)
    ell = jnp.log(rowsum + rowmax * (factor - 1.0))
    acc_ref[0] += jnp.sum(ell)
    acc_ref[1] += jnp.sum(ell * ell)

    @pl.when(i == pl.num_programs(0) - 1)
    def _fin():
        logf = jnp.log(factor)
        a = jnp.float32(_A)
        b = jnp.float32(_B)
        out_ref[0] = (a * acc_ref[1] - 2.0 * logf * acc_ref[0] + b * logf * logf) / (a * b)


@functools.partial(jax.jit, static_argnames=("interpret",))
def _run(action, label_i32, interpret=False):
    grid = _B // _BLOCK_ROWS
    out = pl.pallas_call(
        _loss_kernel,
        grid=(grid,),
        in_specs=[
            pl.BlockSpec(memory_space=pltpu.SMEM),
            pl.BlockSpec((_BLOCK_ROWS, _A), lambda i: (i, 0)),
        ],
        out_specs=pl.BlockSpec(memory_space=pltpu.SMEM),
        out_shape=jax.ShapeDtypeStruct((1,), jnp.float32),
        scratch_shapes=[pltpu.SMEM((2,), jnp.float32)],
        interpret=interpret,
    )(label_i32, action)
    return out[0]


def kernel(action, label):
    return _run(action, label.astype(jnp.int32))
